# Initial kernel scaffold; baseline (speedup 1.0000x reference)
import jax
import jax.numpy as jnp
from jax.experimental import pallas as pl


def kernel(x, embedding):
    raise NotImplementedError("write your pallas kernel here")


# speedup vs baseline: 1.6838x; 1.6838x over previous
import functools

import jax
import jax.numpy as jnp
from jax import lax
from jax.experimental import pallas as pl
from jax.experimental.pallas import tpu as pltpu
from jax.experimental.pallas import tpu_sc as plsc

EMBED_DIM = 64
NC = 2
NS = 16
NW = NC * NS
CHUNK = 128


@functools.lru_cache(maxsize=None)
def _build(B: int, V: int, D: int):
    rows_w = B // NW
    nchunk = rows_w // CHUNK

    mesh = plsc.VectorSubcoreMesh(core_axis_name="c", subcore_axis_name="s")

    @functools.partial(
        pl.kernel,
        mesh=mesh,
        out_type=jax.ShapeDtypeStruct((B, D), jnp.float32),
        compiler_params=pltpu.CompilerParams(use_tc_tiling_on_sc=False),
        scratch_types=[
            pltpu.VMEM((nchunk, CHUNK), jnp.int32),
            pltpu.VMEM((CHUNK, D), jnp.float32),
            pltpu.SemaphoreType.DMA,
        ],
    )
    def gather_kernel(table_hbm, idx_hbm, out_hbm, idx_v, rows_v, gsem):
        wid = lax.axis_index("s") * NC + lax.axis_index("c")
        base_chunk = wid * nchunk
        pltpu.sync_copy(idx_hbm.at[pl.ds(base_chunk, nchunk)], idx_v)

        def step(g, carry):
            pltpu.async_copy(table_hbm.at[idx_v.at[g]], rows_v, gsem).wait()
            pltpu.sync_copy(
                rows_v, out_hbm.at[pl.ds((base_chunk + g) * CHUNK, CHUNK)])
            return carry

        lax.fori_loop(0, nchunk, step, 0)

    return gather_kernel


def kernel(x, embedding):
    bsz, hist = x.shape
    B = bsz * hist
    V, D = embedding.shape
    idx = x.reshape(B // CHUNK, CHUNK)
    out = _build(B, V, D)(embedding, idx)
    return out.reshape(bsz, hist, D)


# speedup vs baseline: 1.8735x; 1.1127x over previous
import functools

import jax
import jax.numpy as jnp
from jax import lax
from jax.experimental import pallas as pl
from jax.experimental.pallas import tpu as pltpu
from jax.experimental.pallas import tpu_sc as plsc

EMBED_DIM = 64
NC = 2
NS = 16
NW = NC * NS
CHUNK = 128
K = 2
NBUF = 4
PF = 2
ROWS_G = K * CHUNK


@functools.lru_cache(maxsize=None)
def _build(B: int, V: int, D: int):
    rows_w = B // NW
    nchunk = rows_w // CHUNK
    ngroup = nchunk // K
    nblk = ngroup // NBUF
    assert ngroup % NBUF == 0 and nblk >= 2

    mesh = plsc.VectorSubcoreMesh(core_axis_name="c", subcore_axis_name="s")

    @functools.partial(
        pl.kernel,
        mesh=mesh,
        out_type=jax.ShapeDtypeStruct((B, D), jnp.float32),
        compiler_params=pltpu.CompilerParams(use_tc_tiling_on_sc=False),
        scratch_types=[
            pltpu.VMEM((nchunk, CHUNK), jnp.int32),
            pltpu.VMEM((NBUF, ROWS_G, D), jnp.float32),
        ]
        + [pltpu.SemaphoreType.DMA] * (2 * NBUF),
    )
    def gather_kernel(table_hbm, idx_hbm, out_hbm, idx_v, rows_v, *sems):
        gsems, osems = sems[:NBUF], sems[NBUF:]
        wid = lax.axis_index("s") * NC + lax.axis_index("c")
        base_chunk = wid * nchunk
        base_row = base_chunk * CHUNK
        pltpu.sync_copy(idx_hbm.at[pl.ds(base_chunk, nchunk)], idx_v)

        def gather_copy(g, b, j):
            return pltpu.make_async_copy(
                table_hbm.at[idx_v.at[g * K + j]],
                rows_v.at[b].at[pl.ds(j * CHUNK, CHUNK)],
                gsems[b])

        def out_copy(g, b):
            return pltpu.make_async_copy(
                rows_v.at[b],
                out_hbm.at[pl.ds(base_row + g * ROWS_G, ROWS_G)],
                osems[b])

        def fire_gather(g, b):
            for j in range(K):
                gather_copy(g, b, j).start()

        def drain_gather(g, b):
            for j in range(K):
                gather_copy(g, b, j).wait()

        def iter_body(g, b, *, first_blk):
            drain_gather(g, b)
            out_copy(g, b).start()
            p = g + PF
            bp = (b + PF) % NBUF
            if not first_blk:
                out_copy(p - NBUF, bp).wait()
            fire_gather(p, bp)

        for b in range(PF):
            fire_gather(b, b)
        for b in range(NBUF):
            iter_body(b, b, first_blk=(b < NBUF - PF))

        def blk(r, carry):
            for b in range(NBUF):
                iter_body(r * NBUF + b, b, first_blk=False)
            return carry

        lax.fori_loop(1, nblk - 1, blk, 0)

        for b in range(NBUF):
            g = (nblk - 1) * NBUF + b
            drain_gather(g, b)
            out_copy(g, b).start()
            p = g + PF
            if p < ngroup:
                bp = (b + PF) % NBUF
                out_copy(p - NBUF, bp).wait()
                fire_gather(p, bp)
        for b in range(NBUF):
            out_copy((nblk - 1) * NBUF + b, b).wait()

    return gather_kernel


def kernel(x, embedding):
    bsz, hist = x.shape
    B = bsz * hist
    V, D = embedding.shape
    idx = x.reshape(B // CHUNK, CHUNK)
    out = _build(B, V, D)(embedding, idx)
    return out.reshape(bsz, hist, D)


# speedup vs baseline: 1.8832x; 1.0052x over previous
import functools

import jax
import jax.numpy as jnp
from jax import lax
from jax.experimental import pallas as pl
from jax.experimental.pallas import tpu as pltpu
from jax.experimental.pallas import tpu_sc as plsc

NC = 2
NS = 16
NW = NC * NS
GB = 4
NBUF = 4
PF = 2


@functools.lru_cache(maxsize=None)
def _build(BATCH: int, H: int, V: int, D: int):
    bat_w = BATCH // NW
    ngroup = bat_w // GB
    nblk = ngroup // NBUF
    assert ngroup % NBUF == 0 and nblk >= 2

    mesh = plsc.VectorSubcoreMesh(core_axis_name="c", subcore_axis_name="s")

    @functools.partial(
        pl.kernel,
        mesh=mesh,
        out_type=jax.ShapeDtypeStruct((BATCH, H, D), jnp.float32),
        compiler_params=pltpu.CompilerParams(use_tc_tiling_on_sc=False),
        scratch_types=[
            pltpu.VMEM((bat_w, H), jnp.int32),
            pltpu.VMEM((NBUF, GB, H, D), jnp.float32),
        ]
        + [pltpu.SemaphoreType.DMA] * (2 * NBUF),
    )
    def gather_kernel(table_hbm, idx_hbm, out_hbm, idx_v, rows_v, *sems):
        gsems, osems = sems[:NBUF], sems[NBUF:]
        wid = lax.axis_index("s") * NC + lax.axis_index("c")
        base_bat = wid * bat_w
        pltpu.sync_copy(idx_hbm.at[pl.ds(base_bat, bat_w)], idx_v)

        def gather_copy(g, b, j):
            return pltpu.make_async_copy(
                table_hbm.at[idx_v.at[g * GB + j]],
                rows_v.at[b].at[j],
                gsems[b])

        def out_copy(g, b):
            return pltpu.make_async_copy(
                rows_v.at[b],
                out_hbm.at[pl.ds(base_bat + g * GB, GB)],
                osems[b])

        def fire_gather(g, b):
            for j in range(GB):
                gather_copy(g, b, j).start()

        def drain_gather(g, b):
            for j in range(GB):
                gather_copy(g, b, j).wait()

        def iter_body(g, b, *, first_blk):
            drain_gather(g, b)
            out_copy(g, b).start()
            p = g + PF
            bp = (b + PF) % NBUF
            if not first_blk:
                out_copy(p - NBUF, bp).wait()
            fire_gather(p, bp)

        for b in range(PF):
            fire_gather(b, b)
        for b in range(NBUF):
            iter_body(b, b, first_blk=(b < NBUF - PF))

        def blk(r, carry):
            for b in range(NBUF):
                iter_body(r * NBUF + b, b, first_blk=False)
            return carry

        lax.fori_loop(1, nblk - 1, blk, 0)

        for b in range(NBUF):
            g = (nblk - 1) * NBUF + b
            drain_gather(g, b)
            out_copy(g, b).start()
            p = g + PF
            if p < ngroup:
                bp = (b + PF) % NBUF
                out_copy(p - NBUF, bp).wait()
                fire_gather(p, bp)
        for b in range(NBUF):
            out_copy((nblk - 1) * NBUF + b, b).wait()

    return gather_kernel


def kernel(x, embedding):
    bsz, hist = x.shape
    V, D = embedding.shape
    return _build(bsz, hist, V, D)(embedding, x)


# speedup vs baseline: 1.9707x; 1.0464x over previous
import functools

import jax
import jax.numpy as jnp
from jax import lax
from jax.experimental import pallas as pl
from jax.experimental.pallas import tpu as pltpu
from jax.experimental.pallas import tpu_sc as plsc

NC = 2
NS = 16
NW = NC * NS
GB = 4
NBUF = 4
PF = 2


@functools.lru_cache(maxsize=None)
def _build(BATCH: int, H: int, V: int, D: int):
    bat_w = BATCH // NW
    ngroup = bat_w // GB
    nblk = ngroup // NBUF
    assert ngroup % NBUF == 0 and nblk >= 2

    mesh = plsc.VectorSubcoreMesh(core_axis_name="c", subcore_axis_name="s")

    @functools.partial(
        pl.kernel,
        mesh=mesh,
        out_type=jax.ShapeDtypeStruct((BATCH, H, D), jnp.float32),
        compiler_params=pltpu.CompilerParams(use_tc_tiling_on_sc=False),
        scratch_types=[
            pltpu.VMEM((bat_w, H), jnp.int32),
            pltpu.VMEM((NBUF, GB, H, D), jnp.float32),
        ]
        + [pltpu.SemaphoreType.DMA] * (2 * NBUF),
    )
    def gather_kernel(table_hbm, idx_hbm, out_hbm, idx_v, rows_v, *sems):
        gsems, osems = sems[:NBUF], sems[NBUF:]
        wid = lax.axis_index("s") * NC + lax.axis_index("c")
        base_bat = wid * bat_w
        pltpu.sync_copy(idx_hbm.at[pl.ds(base_bat, bat_w)], idx_v)

        def gather_copy(g, b, j):
            return pltpu.make_async_copy(
                table_hbm.at[idx_v.at[g * GB + j]],
                rows_v.at[b].at[j],
                gsems[b])

        def out_copy(g, b):
            return pltpu.make_async_copy(
                rows_v.at[b],
                out_hbm.at[pl.ds(base_bat + g * GB, GB)],
                osems[b])

        def fire_gather(g, b):
            for j in range(GB):
                gather_copy(g, b, j).start()

        def drain_gather(g, b):
            for j in range(GB):
                gather_copy(g, b, j).wait()

        def iter_body(g, b, *, first_blk):
            drain_gather(g, b)
            out_copy(g, b).start()
            p = g + PF
            bp = (b + PF) % NBUF
            if not first_blk:
                out_copy(p - NBUF, bp).wait()
            fire_gather(p, bp)

        for b in range(PF):
            fire_gather(b, b)
        for b in range(NBUF):
            iter_body(b, b, first_blk=(b < NBUF - PF))

        def blk(r, carry):
            for b in range(NBUF):
                iter_body(r * NBUF + b, b, first_blk=False)
            return carry

        lax.fori_loop(1, nblk - 1, blk, 0)

        for b in range(NBUF):
            g = (nblk - 1) * NBUF + b
            drain_gather(g, b)
            out_copy(g, b).start()
            p = g + PF
            if p < ngroup:
                bp = (b + PF) % NBUF
                out_copy(p - NBUF, bp).wait()
                fire_gather(p, bp)
        for b in range(NBUF):
            out_copy((nblk - 1) * NBUF + b, b).wait()

    return gather_kernel


def kernel(x, embedding):
    bsz, hist = x.shape
    V, D = embedding.shape
    tab2 = jnp.pad(embedding, ((0, 0), (0, 128 - D))).reshape(2 * V, D)
    return _build(bsz, hist, 2 * V, D)(tab2, x * 2)
